# baseline (speedup 1.0000x reference)
import functools

import jax
import jax.numpy as jnp
from jax import lax
from jax.experimental import pallas as pl
from jax.experimental.pallas import tpu as pltpu
from jax.experimental.pallas import tpu_sc as plsc

B, H, W, C = 4, 224, 224, 96
NPIX = B * H * W
NW = 32
PPW = NPIX // NW
K = 32
NCHUNK = PPW // K
L = 16

_mesh = plsc.VectorSubcoreMesh(core_axis_name="c", subcore_axis_name="s")


@functools.partial(
    pl.kernel,
    mesh=_mesh,
    out_type=jax.ShapeDtypeStruct((NPIX, C), jnp.float32),
    scratch_types=[
        pltpu.VMEM((PPW,), jnp.float32),
        pltpu.VMEM((PPW,), jnp.float32),
        pltpu.VMEM((4 * K,), jnp.int32),
        pltpu.VMEM((4 * K, C), jnp.float32),
        pltpu.VMEM((K, C), jnp.float32),
        pltpu.SemaphoreType.DMA,
    ],
    compiler_params=pltpu.CompilerParams(
        needs_layout_passes=False, use_tc_tiling_on_sc=False),
)
def _resample_sc(table_hbm, coords_hbm, out_hbm, ys_v, xs_v, idx_v, rows_v,
                 out_v, sem):
    wid = lax.axis_index("s") * 2 + lax.axis_index("c")
    pbase = wid * PPW
    boff = (pbase // (H * W)) * (H * W)
    pltpu.sync_copy(coords_hbm.at[0, pl.ds(pbase, PPW)], ys_v)
    pltpu.sync_copy(coords_hbm.at[1, pl.ds(pbase, PPW)], xs_v)
    lane = lax.iota(jnp.int32, L)

    def chunk_body(j, carry):
        weights = []
        for h in range(K // L):
            y = ys_v[pl.ds(j * K + h * L, L)]
            x = xs_v[pl.ds(j * K + h * L, L)]
            y0 = jnp.clip(y.astype(jnp.int32), 0, H - 2)
            x0 = jnp.clip(x.astype(jnp.int32), 0, W - 2)
            wy = y - y0.astype(jnp.float32)
            wx = x - x0.astype(jnp.float32)
            base = boff + y0 * W + x0
            idx_v[pl.ds(0 * K + h * L, L)] = base
            idx_v[pl.ds(1 * K + h * L, L)] = base + 1
            idx_v[pl.ds(2 * K + h * L, L)] = base + W
            idx_v[pl.ds(3 * K + h * L, L)] = base + W + 1
            weights.append((wy, wx))

        pltpu.async_copy(table_hbm.at[idx_v], rows_v, sem).wait()

        for h in range(K // L):
            wy, wx = weights[h]
            w00 = (1.0 - wy) * (1.0 - wx)
            w01 = (1.0 - wy) * wx
            w10 = wy * (1.0 - wx)
            w11 = wy * wx
            prow = h * L + lane
            r0 = prow
            r1 = prow + K
            r2 = prow + 2 * K
            r3 = prow + 3 * K

            def cbody(c, _, w00=w00, w01=w01, w10=w10, w11=w11,
                      r0=r0, r1=r1, r2=r2, r3=r3, prow=prow):
                col = jnp.full((L,), c, jnp.int32)
                a = plsc.load_gather(rows_v, [r0, col])
                b = plsc.load_gather(rows_v, [r1, col])
                cc = plsc.load_gather(rows_v, [r2, col])
                d = plsc.load_gather(rows_v, [r3, col])
                o = w00 * a + w01 * b + w10 * cc + w11 * d
                plsc.store_scatter(out_v, [prow, col], o)
                return _

            lax.fori_loop(0, C, cbody, 0)

        pltpu.sync_copy(out_v, out_hbm.at[pl.ds(pbase + j * K, K)])
        return carry

    lax.fori_loop(0, NCHUNK, chunk_body, 0)


def kernel(inputs, sample_coords):
    table = inputs.reshape(B * H * W, C)
    coords = jnp.moveaxis(sample_coords.reshape(NPIX, 2), -1, 0)
    out = _resample_sc(table, coords)
    return out.reshape(B, H, W, C)


# speedup vs baseline: 1.0737x; 1.0737x over previous
import functools

import jax
import jax.numpy as jnp
from jax import lax
from jax.experimental import pallas as pl
from jax.experimental.pallas import tpu as pltpu
from jax.experimental.pallas import tpu_sc as plsc

B, H, W, C = 4, 224, 224, 96
NPIX = B * H * W
NW = 32
PPW = NPIX // NW
K = 32
NCHUNK = PPW // K
L = 16

_mesh = plsc.VectorSubcoreMesh(core_axis_name="c", subcore_axis_name="s")


@functools.partial(
    pl.kernel,
    mesh=_mesh,
    out_type=jax.ShapeDtypeStruct((NPIX, C), jnp.float32),
    scratch_types=[
        pltpu.VMEM((PPW,), jnp.float32),
        pltpu.VMEM((PPW,), jnp.float32),
        pltpu.VMEM((4 * K,), jnp.int32),
        pltpu.VMEM((4 * K,), jnp.int32),
        pltpu.VMEM((4 * K,), jnp.float32),
        pltpu.VMEM((4 * K,), jnp.float32),
        pltpu.VMEM((4 * K, C), jnp.float32),
        pltpu.VMEM((4 * K, C), jnp.float32),
        pltpu.VMEM((K, C), jnp.float32),
        pltpu.VMEM((K, C), jnp.float32),
        pltpu.SemaphoreType.DMA,
        pltpu.SemaphoreType.DMA,
        pltpu.SemaphoreType.DMA,
        pltpu.SemaphoreType.DMA,
    ],
    compiler_params=pltpu.CompilerParams(
        needs_layout_passes=False, use_tc_tiling_on_sc=False),
)
def _resample_sc(table_hbm, coords_hbm, out_hbm, ys_v, xs_v, idx0, idx1,
                 w0, w1, rows0, rows1, out0, out1, gsem0, gsem1, osem0,
                 osem1):
    idx_s = (idx0, idx1)
    w_s = (w0, w1)
    rows_s = (rows0, rows1)
    out_s = (out0, out1)
    gsem_s = (gsem0, gsem1)
    osem_s = (osem0, osem1)

    wid = lax.axis_index("s") * 2 + lax.axis_index("c")
    pbase = wid * PPW
    boff = (pbase // (H * W)) * (H * W)
    pltpu.sync_copy(coords_hbm.at[0, pl.ds(pbase, PPW)], ys_v)
    pltpu.sync_copy(coords_hbm.at[1, pl.ds(pbase, PPW)], xs_v)
    lane = lax.iota(jnp.int32, L)

    def prep(j, b):
        for h in range(K // L):
            y = ys_v[pl.ds(j * K + h * L, L)]
            x = xs_v[pl.ds(j * K + h * L, L)]
            y0 = jnp.clip(y.astype(jnp.int32), 0, H - 2)
            x0 = jnp.clip(x.astype(jnp.int32), 0, W - 2)
            wy = y - y0.astype(jnp.float32)
            wx = x - x0.astype(jnp.float32)
            base = boff + y0 * W + x0
            idx_s[b][pl.ds(0 * K + h * L, L)] = base
            idx_s[b][pl.ds(1 * K + h * L, L)] = base + 1
            idx_s[b][pl.ds(2 * K + h * L, L)] = base + W
            idx_s[b][pl.ds(3 * K + h * L, L)] = base + W + 1
            w_s[b][pl.ds(0 * K + h * L, L)] = (1.0 - wy) * (1.0 - wx)
            w_s[b][pl.ds(1 * K + h * L, L)] = (1.0 - wy) * wx
            w_s[b][pl.ds(2 * K + h * L, L)] = wy * (1.0 - wx)
            w_s[b][pl.ds(3 * K + h * L, L)] = wy * wx
        pltpu.make_async_copy(
            table_hbm.at[idx_s[b]], rows_s[b], gsem_s[b]).start()

    def blend(b):
        for h in range(K // L):
            w00 = w_s[b][pl.ds(0 * K + h * L, L)]
            w01 = w_s[b][pl.ds(1 * K + h * L, L)]
            w10 = w_s[b][pl.ds(2 * K + h * L, L)]
            w11 = w_s[b][pl.ds(3 * K + h * L, L)]
            prow = h * L + lane
            r0 = prow
            r1 = prow + K
            r2 = prow + 2 * K
            r3 = prow + 3 * K

            def cbody(c, _, w00=w00, w01=w01, w10=w10, w11=w11,
                      r0=r0, r1=r1, r2=r2, r3=r3, prow=prow):
                col = jnp.full((L,), c, jnp.int32)
                a = plsc.load_gather(rows_s[b], [r0, col])
                bb = plsc.load_gather(rows_s[b], [r1, col])
                cc = plsc.load_gather(rows_s[b], [r2, col])
                d = plsc.load_gather(rows_s[b], [r3, col])
                o = w00 * a + w01 * bb + w10 * cc + w11 * d
                plsc.store_scatter(out_s[b], [prow, col], o)
                return _

            lax.fori_loop(0, C, cbody, 0, unroll=8)

    prep(0, 0)
    prep(1, 1)

    def chunk_pair(g, carry):
        for b in range(2):
            j = g * 2 + b
            pltpu.make_async_copy(
                table_hbm.at[idx_s[b]], rows_s[b], gsem_s[b]).wait()

            @pl.when(j >= 2)
            def _wait_out(b=b, j=j):
                pltpu.make_async_copy(
                    out_s[b], out_hbm.at[pl.ds(pbase + (j - 2) * K, K)],
                    osem_s[b]).wait()

            blend(b)
            pltpu.make_async_copy(
                out_s[b], out_hbm.at[pl.ds(pbase + j * K, K)],
                osem_s[b]).start()

            @pl.when(j + 2 < NCHUNK)
            def _prep_next(b=b, j=j):
                prep(j + 2, b)
        return carry

    lax.fori_loop(0, NCHUNK // 2, chunk_pair, 0)

    for b in range(2):
        pltpu.make_async_copy(
            out_s[b],
            out_hbm.at[pl.ds(pbase + (NCHUNK - 2 + b) * K, K)],
            osem_s[b]).wait()


def kernel(inputs, sample_coords):
    table = inputs.reshape(B * H * W, C)
    coords = jnp.moveaxis(sample_coords.reshape(NPIX, 2), -1, 0)
    out = _resample_sc(table, coords)
    return out.reshape(B, H, W, C)
